# baseline (speedup 1.0000x reference)
import functools

import jax
import jax.numpy as jnp
from jax import lax
from jax.experimental import pallas as pl
from jax.experimental.pallas import tpu as pltpu
from jax.experimental.pallas import tpu_sc as plsc

_N_CLS = 1000
_N_CTX = 16
_D = 512
_L = 77
_K = 2
_NSLOT = 3

_NC = 2
_NS = 16
_NW = _NC * _NS
_CPW = 32
_SPLICE_END = 20 + _N_CTX - 1


def _body(emb, ctx, pfx, out, buf, ctx_v, pfx_v, psm, *sems):
    w = lax.axis_index("s") * _NC + lax.axis_index("c")
    c0 = w * _CPW
    cw = jnp.minimum(_N_CLS - c0, _CPW)
    srs = sems[:_NSLOT]
    sws = sems[_NSLOT:]
    nfull = _L // _K
    rem = _L - nfull * _K

    pltpu.sync_copy(ctx, ctx_v)
    pltpu.sync_copy(pfx.at[pl.ds(c0, _CPW)], pfx_v)
    pv0 = pfx_v[pl.ds(0, 16)]
    pv1 = pfx_v[pl.ds(16, 16)]
    for j in range(16):
        psm[j] = pv0[j]
        psm[j + 16] = pv1[j]

    def run(width):
        def fire_read(r0, k, slot):
            pltpu.async_copy(emb.at[pl.ds(r0, k), pl.ds(c0, width)],
                             buf.at[slot, pl.ds(0, k), pl.ds(0, width)],
                             srs[slot])

        def wait_read(k, slot):
            pltpu.make_async_copy(emb.at[pl.ds(0, k), pl.ds(c0, width)],
                                  buf.at[slot, pl.ds(0, k), pl.ds(0, width)],
                                  srs[slot]).wait()

        def fire_write(r0, k, slot):
            pltpu.async_copy(buf.at[slot, pl.ds(0, k), pl.ds(0, width)],
                             out.at[pl.ds(r0, k), pl.ds(c0, width)],
                             sws[slot])

        def wait_write(k, slot):
            pltpu.make_async_copy(buf.at[slot, pl.ds(0, k), pl.ds(0, width)],
                                  out.at[pl.ds(0, k), pl.ds(c0, width)],
                                  sws[slot]).wait()

        def splice(r0, slot):
            def cls(t, carry):
                p = psm[t]
                for jr in range(_K):
                    rrel = r0 + jr - p

                    @pl.when((rrel >= 0) & (rrel < _N_CTX))
                    def _():
                        for cc in range(0, _D, 16):
                            buf[slot, jr, t, pl.ds(cc, 16)] = (
                                ctx_v[rrel, pl.ds(cc, 16)])

                return carry

            lax.fori_loop(0, width, cls, None)

        for slot in range(_NSLOT):
            fire_read(slot * _K, _K, slot)

        def step(g, carry):
            base = g * _NSLOT
            for slot in range(_NSLOT):
                ci = base + slot
                r0 = ci * _K
                wait_read(_K, slot)

                @pl.when(r0 <= _SPLICE_END)
                def _():
                    splice(r0, slot)

                fire_write(r0, _K, slot)

                @pl.when(ci + _NSLOT < nfull)
                def _():
                    wait_write(_K, slot)
                    fire_read((ci + _NSLOT) * _K, _K, slot)

            return carry

        lax.fori_loop(0, nfull // _NSLOT, step, None)
        done = (nfull // _NSLOT) * _NSLOT
        for slot in range(nfull - done):
            r0 = (done + slot) * _K
            wait_read(_K, slot)
            fire_write(r0, _K, slot)
        rslot = nfull - done
        wait_write(_K, rslot)
        fire_read(nfull * _K, rem, rslot)
        wait_read(rem, rslot)
        fire_write(nfull * _K, rem, rslot)
        for slot in range(nfull - done):
            wait_write(_K, slot)
        wait_write(rem, rslot)

    @pl.when(cw >= _CPW)
    def _():
        run(_CPW)

    @pl.when(cw < _CPW)
    def _():
        run(_N_CLS - (_NW - 1) * _CPW)


@functools.partial(
    pl.kernel,
    out_type=jax.ShapeDtypeStruct((_L, _N_CLS, _D), jnp.float32),
    mesh=plsc.VectorSubcoreMesh(core_axis_name="c", subcore_axis_name="s"),
    scratch_types=[
        pltpu.VMEM((_NSLOT, _K, _CPW, _D), jnp.float32),
        pltpu.VMEM((_N_CTX, _D), jnp.float32),
        pltpu.VMEM((_CPW,), jnp.int32),
        pltpu.SMEM((_CPW,), jnp.int32),
        pltpu.SemaphoreType.DMA,
        pltpu.SemaphoreType.DMA,
        pltpu.SemaphoreType.DMA,
        pltpu.SemaphoreType.DMA,
        pltpu.SemaphoreType.DMA,
        pltpu.SemaphoreType.DMA,
    ],
)
def _splice_kernel(emb, ctx, pfx, out, buf, ctx_v, pfx_v, psm, *sems):
    _body(emb, ctx, pfx, out, buf, ctx_v, pfx_v, psm, *sems)


def kernel(origin_text_embedding, ctx, prefix_index):
    emb_t = jnp.transpose(origin_text_embedding, (1, 0, 2))
    pfx = jnp.pad(prefix_index, (0, _NW * _CPW - _N_CLS))
    out_t = _splice_kernel(emb_t, ctx, pfx)
    return jnp.transpose(out_t, (1, 0, 2))


# speedup vs baseline: 1.0055x; 1.0055x over previous
import functools

import jax
import jax.numpy as jnp
from jax import lax
from jax.experimental import pallas as pl
from jax.experimental.pallas import tpu as pltpu
from jax.experimental.pallas import tpu_sc as plsc

_N_CLS = 1000
_N_CTX = 16
_D = 512
_L = 77
_K = 3
_NSLOT = 2

_NC = 2
_NS = 16
_NW = _NC * _NS
_CPW = 32
_SPLICE_END = 20 + _N_CTX - 1


def _body(emb, ctx, pfx, out, buf, ctx_v, pfx_v, psm, sr0, sr1, sw0, sw1):
    w = lax.axis_index("s") * _NC + lax.axis_index("c")
    c0 = w * _CPW
    cw = jnp.minimum(_N_CLS - c0, _CPW)
    srs = (sr0, sr1)
    sws = (sw0, sw1)
    nfull = _L // _K
    rem = _L - nfull * _K

    pltpu.sync_copy(ctx, ctx_v)
    pltpu.sync_copy(pfx.at[pl.ds(c0, _CPW)], pfx_v)
    pv0 = pfx_v[pl.ds(0, 16)]
    pv1 = pfx_v[pl.ds(16, 16)]
    for j in range(16):
        psm[j] = pv0[j]
        psm[j + 16] = pv1[j]

    def run(width):
        def fire_read(r0, k, slot):
            pltpu.async_copy(emb.at[pl.ds(r0, k), pl.ds(c0, width)],
                             buf.at[slot, pl.ds(0, k), pl.ds(0, width)],
                             srs[slot])

        def wait_read(k, slot):
            pltpu.make_async_copy(emb.at[pl.ds(0, k), pl.ds(c0, width)],
                                  buf.at[slot, pl.ds(0, k), pl.ds(0, width)],
                                  srs[slot]).wait()

        def fire_write(r0, k, slot):
            pltpu.async_copy(buf.at[slot, pl.ds(0, k), pl.ds(0, width)],
                             out.at[pl.ds(r0, k), pl.ds(c0, width)],
                             sws[slot])

        def wait_write(k, slot):
            pltpu.make_async_copy(buf.at[slot, pl.ds(0, k), pl.ds(0, width)],
                                  out.at[pl.ds(0, k), pl.ds(c0, width)],
                                  sws[slot]).wait()

        def splice(r0, slot):
            def cls(t, carry):
                p = psm[t]
                for jr in range(_K):
                    rrel = r0 + jr - p

                    @pl.when((rrel >= 0) & (rrel < _N_CTX))
                    def _():
                        for cc in range(0, _D, 16):
                            buf[slot, jr, t, pl.ds(cc, 16)] = (
                                ctx_v[rrel, pl.ds(cc, 16)])

                return carry

            lax.fori_loop(0, width, cls, None)

        for slot in range(_NSLOT):
            fire_read(slot * _K, _K, slot)

        def step(g, carry):
            base = g * _NSLOT
            for slot in range(_NSLOT):
                ci = base + slot
                r0 = ci * _K
                wait_read(_K, slot)

                @pl.when(r0 <= _SPLICE_END)
                def _():
                    splice(r0, slot)

                fire_write(r0, _K, slot)

                @pl.when(ci + _NSLOT < nfull)
                def _():
                    wait_write(_K, slot)
                    fire_read((ci + _NSLOT) * _K, _K, slot)

            return carry

        lax.fori_loop(0, nfull // _NSLOT, step, None)
        wait_read(_K, 0)
        fire_write((nfull - 1) * _K, _K, 0)
        wait_write(_K, 1)
        fire_read(nfull * _K, rem, 1)
        wait_read(rem, 1)
        fire_write(nfull * _K, rem, 1)
        wait_write(_K, 0)
        wait_write(rem, 1)

    @pl.when(cw >= _CPW)
    def _():
        run(_CPW)

    @pl.when(cw < _CPW)
    def _():
        run(_N_CLS - (_NW - 1) * _CPW)


@functools.partial(
    pl.kernel,
    out_type=jax.ShapeDtypeStruct((_L, _N_CLS, _D), jnp.float32),
    mesh=plsc.VectorSubcoreMesh(core_axis_name="c", subcore_axis_name="s"),
    scratch_types=[
        pltpu.VMEM((_NSLOT, _K, _CPW, _D), jnp.float32),
        pltpu.VMEM((_N_CTX, _D), jnp.float32),
        pltpu.VMEM((_CPW,), jnp.int32),
        pltpu.SMEM((_CPW,), jnp.int32),
        pltpu.SemaphoreType.DMA,
        pltpu.SemaphoreType.DMA,
        pltpu.SemaphoreType.DMA,
        pltpu.SemaphoreType.DMA,
    ],
)
def _splice_kernel(emb, ctx, pfx, out, buf, ctx_v, pfx_v, psm, sr0, sr1, sw0, sw1):
    _body(emb, ctx, pfx, out, buf, ctx_v, pfx_v, psm, sr0, sr1, sw0, sw1)


def kernel(origin_text_embedding, ctx, prefix_index):
    emb_t = jnp.transpose(origin_text_embedding, (1, 0, 2))
    pfx = jnp.pad(prefix_index, (0, _NW * _CPW - _N_CLS))
    out_t = _splice_kernel(emb_t, ctx, pfx)
    return jnp.transpose(out_t, (1, 0, 2))


# speedup vs baseline: 1.0058x; 1.0004x over previous
import functools

import jax
import jax.numpy as jnp
from jax import lax
from jax.experimental import pallas as pl
from jax.experimental.pallas import tpu as pltpu
from jax.experimental.pallas import tpu_sc as plsc

_N_CLS = 1000
_N_CTX = 16
_D = 512
_L = 77
_KS = (3, 2, 2)
_OFF = (0, 3, 5)
_CYC = 7
_NCYC = _L // _CYC
_NSLOT = len(_KS)

_NC = 2
_NS = 16
_NW = _NC * _NS
_CPW = 32
_SPLICE_END = 20 + _N_CTX - 1


def _body(emb, ctx, pfx, out, buf0, buf1, buf2, ctx_v, pfx_v, psm, *sems):
    bufs = (buf0, buf1, buf2)
    w = lax.axis_index("s") * _NC + lax.axis_index("c")
    c0 = w * _CPW
    cw = jnp.minimum(_N_CLS - c0, _CPW)
    srs = sems[:_NSLOT]
    sws = sems[_NSLOT:]

    pltpu.sync_copy(ctx, ctx_v)
    pltpu.sync_copy(pfx.at[pl.ds(c0, _CPW)], pfx_v)
    pv0 = pfx_v[pl.ds(0, 16)]
    pv1 = pfx_v[pl.ds(16, 16)]
    for j in range(16):
        psm[j] = pv0[j]
        psm[j + 16] = pv1[j]

    def run(width):
        def fire_read(r0, slot):
            pltpu.async_copy(emb.at[pl.ds(r0, _KS[slot]), pl.ds(c0, width)],
                             bufs[slot].at[pl.ds(0, _KS[slot]), pl.ds(0, width)],
                             srs[slot])

        def wait_read(slot):
            pltpu.make_async_copy(
                emb.at[pl.ds(0, _KS[slot]), pl.ds(c0, width)],
                bufs[slot].at[pl.ds(0, _KS[slot]), pl.ds(0, width)],
                srs[slot]).wait()

        def fire_write(r0, slot):
            pltpu.async_copy(bufs[slot].at[pl.ds(0, _KS[slot]), pl.ds(0, width)],
                             out.at[pl.ds(r0, _KS[slot]), pl.ds(c0, width)],
                             sws[slot])

        def wait_write(slot):
            pltpu.make_async_copy(
                bufs[slot].at[pl.ds(0, _KS[slot]), pl.ds(0, width)],
                out.at[pl.ds(0, _KS[slot]), pl.ds(c0, width)],
                sws[slot]).wait()

        def splice(r0, slot):
            def cls(t, carry):
                p = psm[t]
                for jr in range(_KS[slot]):
                    rrel = r0 + jr - p

                    @pl.when((rrel >= 0) & (rrel < _N_CTX))
                    def _():
                        for cc in range(0, _D, 16):
                            bufs[slot][jr, t, pl.ds(cc, 16)] = (
                                ctx_v[rrel, pl.ds(cc, 16)])

                return carry

            lax.fori_loop(0, width, cls, None)

        for slot in range(_NSLOT):
            fire_read(_OFF[slot], slot)

        def step(g, carry):
            cyc0 = g * _CYC
            for slot in range(_NSLOT):
                r0 = cyc0 + _OFF[slot]
                wait_read(slot)

                @pl.when(r0 <= _SPLICE_END)
                def _():
                    splice(r0, slot)

                fire_write(r0, slot)

                @pl.when(g + 1 < _NCYC)
                def _():
                    wait_write(slot)
                    fire_read(r0 + _CYC, slot)

            return carry

        lax.fori_loop(0, _NCYC, step, None)
        for slot in range(_NSLOT):
            wait_write(slot)

    @pl.when(cw >= _CPW)
    def _():
        run(_CPW)

    @pl.when(cw < _CPW)
    def _():
        run(_N_CLS - (_NW - 1) * _CPW)


@functools.partial(
    pl.kernel,
    out_type=jax.ShapeDtypeStruct((_L, _N_CLS, _D), jnp.float32),
    mesh=plsc.VectorSubcoreMesh(core_axis_name="c", subcore_axis_name="s"),
    scratch_types=[
        pltpu.VMEM((_KS[0], _CPW, _D), jnp.float32),
        pltpu.VMEM((_KS[1], _CPW, _D), jnp.float32),
        pltpu.VMEM((_KS[2], _CPW, _D), jnp.float32),
        pltpu.VMEM((_N_CTX, _D), jnp.float32),
        pltpu.VMEM((_CPW,), jnp.int32),
        pltpu.SMEM((_CPW,), jnp.int32),
        pltpu.SemaphoreType.DMA,
        pltpu.SemaphoreType.DMA,
        pltpu.SemaphoreType.DMA,
        pltpu.SemaphoreType.DMA,
        pltpu.SemaphoreType.DMA,
        pltpu.SemaphoreType.DMA,
    ],
)
def _splice_kernel(emb, ctx, pfx, out, buf0, buf1, buf2, ctx_v, pfx_v, psm, *sems):
    _body(emb, ctx, pfx, out, buf0, buf1, buf2, ctx_v, pfx_v, psm, *sems)


def kernel(origin_text_embedding, ctx, prefix_index):
    emb_t = jnp.transpose(origin_text_embedding, (1, 0, 2))
    pfx = jnp.pad(prefix_index, (0, _NW * _CPW - _N_CLS))
    out_t = _splice_kernel(emb_t, ctx, pfx)
    return jnp.transpose(out_t, (1, 0, 2))


# speedup vs baseline: 1.2156x; 1.2086x over previous
import functools

import jax
import jax.numpy as jnp
from jax import lax
from jax.experimental import pallas as pl
from jax.experimental.pallas import tpu as pltpu
from jax.experimental.pallas import tpu_sc as plsc

_N_CLS = 1000
_N_CTX = 16
_D = 512
_L = 77
_NSLOT = 7
_NCYC = _L // _NSLOT
_SCATTER_CYC = 6

_NC = 2
_NS = 16
_NW = _NC * _NS
_CPW = 32


def _body(emb, ctx, pfx, out, bufs, ctx_v, pfx_v, psm, srs, sws, ssc):
    w = lax.axis_index("s") * _NC + lax.axis_index("c")
    c0 = w * _CPW
    cw = jnp.minimum(_N_CLS - c0, _CPW)

    pltpu.sync_copy(ctx, ctx_v)
    pltpu.sync_copy(pfx.at[pl.ds(c0, _CPW)], pfx_v)
    pv0 = pfx_v[pl.ds(0, 16)]
    pv1 = pfx_v[pl.ds(16, 16)]
    for j in range(16):
        psm[j] = pv0[j]
        psm[j + 16] = pv1[j]

    lanes = lax.iota(jnp.int32, 16)

    def run(width):
        def fire_read(r, slot):
            pltpu.async_copy(emb.at[pl.ds(r * _N_CLS + c0, width)],
                             bufs[slot].at[pl.ds(0, width)], srs[slot])

        def wait_read(slot):
            pltpu.make_async_copy(emb.at[pl.ds(c0, width)],
                                  bufs[slot].at[pl.ds(0, width)],
                                  srs[slot]).wait()

        def fire_write(r, slot):
            pltpu.async_copy(bufs[slot].at[pl.ds(0, width)],
                             out.at[pl.ds(r * _N_CLS + c0, width)], sws[slot])

        def wait_write(slot):
            pltpu.make_async_copy(bufs[slot].at[pl.ds(0, width)],
                                  out.at[pl.ds(c0, width)], sws[slot]).wait()

        def fire_scatters():
            def cls(t, carry):
                p = psm[t]
                idx = (p + lanes) * _N_CLS + (c0 + t)
                pltpu.async_copy(ctx_v, out.at[idx], ssc)
                return carry

            lax.fori_loop(0, width, cls, None)

        def wait_scatters():
            def cls(t, carry):
                idx = lanes * _N_CLS
                pltpu.make_async_copy(ctx_v, out.at[idx], ssc).wait()
                return carry

            lax.fori_loop(0, width, cls, None)

        for slot in range(_NSLOT):
            fire_read(slot, slot)

        def step(g, carry):
            r0 = g * _NSLOT
            for slot in range(_NSLOT):
                r = r0 + slot
                wait_read(slot)
                fire_write(r, slot)

                @pl.when(g + 1 < _NCYC)
                def _():
                    wait_write(slot)
                    fire_read(r + _NSLOT, slot)

            @pl.when(g == _SCATTER_CYC)
            def _():
                fire_scatters()

            return carry

        lax.fori_loop(0, _NCYC, step, None)
        for slot in range(_NSLOT):
            wait_write(slot)
        wait_scatters()

    @pl.when(cw >= _CPW)
    def _():
        run(_CPW)

    @pl.when(cw < _CPW)
    def _():
        run(_N_CLS - (_NW - 1) * _CPW)


@functools.partial(
    pl.kernel,
    out_type=jax.ShapeDtypeStruct((_L * _N_CLS, _D), jnp.float32),
    mesh=plsc.VectorSubcoreMesh(core_axis_name="c", subcore_axis_name="s"),
    scratch_types=(
        [pltpu.VMEM((_CPW, _D), jnp.float32) for _ in range(_NSLOT)]
        + [
            pltpu.VMEM((_N_CTX, _D), jnp.float32),
            pltpu.VMEM((_CPW,), jnp.int32),
            pltpu.SMEM((_CPW,), jnp.int32),
        ]
        + [pltpu.SemaphoreType.DMA for _ in range(2 * _NSLOT + 1)]
    ),
)
def _splice_kernel(emb, ctx, pfx, out, *scratch):
    bufs = scratch[:_NSLOT]
    ctx_v, pfx_v, psm = scratch[_NSLOT:_NSLOT + 3]
    sems = scratch[_NSLOT + 3:]
    srs = sems[:_NSLOT]
    sws = sems[_NSLOT:2 * _NSLOT]
    ssc = sems[2 * _NSLOT]
    _body(emb, ctx, pfx, out, bufs, ctx_v, pfx_v, psm, srs, sws, ssc)


def kernel(origin_text_embedding, ctx, prefix_index):
    emb_t = jnp.transpose(origin_text_embedding, (1, 0, 2))
    emb2 = emb_t.reshape(_L * _N_CLS, _D)
    pfx = jnp.pad(prefix_index, (0, _NW * _CPW - _N_CLS))
    out2 = _splice_kernel(emb2, ctx, pfx)
    return jnp.transpose(out2.reshape(_L, _N_CLS, _D), (1, 0, 2))
